# baseline (speedup 1.0000x reference)
import functools

import jax
import jax.numpy as jnp
from jax import lax
from jax.experimental import pallas as pl
from jax.experimental.pallas import tpu as pltpu
from jax.experimental.pallas import tpu_sc as plsc

N = 200000
B = 2
X = 512
Y = 32
Z = 512
CIN = 64
COUT = 64

TN = 2048
NP = 200704
NTILES = NP // TN

ROWS_TOTAL = B * X * Z
HALF = ROWS_TOTAL // B
NG = 16
CG = COUT // NG
CH = NP // 16
CHB = CH // 128
STRIPE = HALF // 16
DUMMY = HALF
ZROWS = 2048
SUB = 1792
NSUB = CH // SUB
SUBB = SUB // 128


def _phase_a_body(c_ref, f_ref, k_ref, fg_ref, idx_ref):
    c = c_ref[...]
    kidx = c[:, 1:2]
    ix = c[:, 0:1]
    iz = c[:, 2:3]
    ib = c[:, 3:4]
    idx_ref[...] = ib * (X * Z) + ix * Z + iz
    ft = f_ref[...]
    acc = jnp.zeros((TN, COUT), jnp.float32)
    for k in range(Y):
        m = kidx == k
        acc = acc + lax.dot_general(
            jnp.where(m, ft, 0.0), k_ref[k],
            (((1,), (0,)), ((), ())), preferred_element_type=jnp.float32)
    fg_ref[...] = acc.reshape(TN, NG, CG).transpose(1, 0, 2).reshape(
        NG, TN * CG)


def _phase_a(coords_p, feats_p, ker):
    return pl.pallas_call(
        _phase_a_body,
        grid=(NTILES,),
        in_specs=[
            pl.BlockSpec((TN, 4), lambda i: (i, 0)),
            pl.BlockSpec((TN, CIN), lambda i: (i, 0)),
            pl.BlockSpec((Y, CIN, COUT), lambda i: (0, 0, 0)),
        ],
        out_specs=[
            pl.BlockSpec((NG, TN * CG), lambda i: (0, i)),
            pl.BlockSpec((TN, 1), lambda i: (i, 0)),
        ],
        out_shape=[
            jax.ShapeDtypeStruct((NG, NP * CG), jnp.float32),
            jax.ShapeDtypeStruct((NP, 1), jnp.int32),
        ],
    )(coords_p, feats_p, ker)


def _phase_b(fg, idx2d, zeros_src):
    mesh = plsc.VectorSubcoreMesh(core_axis_name="c", subcore_axis_name="s")

    @functools.partial(
        pl.kernel,
        mesh=mesh,
        compiler_params=pltpu.CompilerParams(use_tc_tiling_on_sc=False),
        out_type=jax.ShapeDtypeStruct((NG * ROWS_TOTAL, CG), jnp.float32),
        scratch_types=[
            pltpu.VMEM((CHB, 128), jnp.int32),
            pltpu.VMEM((SUB, CG), jnp.float32),
            pltpu.VMEM((ZROWS, CG), jnp.float32),
            pltpu.VMEM_SHARED((HALF + 16, CG), jnp.float32),
            pltpu.SemaphoreType.DMA,
        ],
    )
    def scatter_kernel(fg_ref, idx_ref, z_ref, out_ref, rows2d, fsub, zbuf,
                       grid_s, sem):
        cid = lax.axis_index("c")
        sid = lax.axis_index("s")
        base = cid * HALF

        pltpu.sync_copy(idx_ref.at[pl.ds(sid * CHB, CHB)], rows2d)
        pltpu.sync_copy(z_ref, zbuf)

        def prep(j, _):
            for q in range(8):
                vec = rows2d[j, pl.ds(q * 16, 16)]
                own = (vec >= base) & (vec < base + HALF)
                rows2d[j, pl.ds(q * 16, 16)] = jnp.where(own, vec - base,
                                                         DUMMY)
            return 0
        lax.fori_loop(0, CHB, prep, 0)

        def do_pass(g, _):
            for k in range(16):
                @pl.when(sid == k)
                def _():
                    for t in range(STRIPE // ZROWS):
                        pltpu.sync_copy(
                            zbuf,
                            grid_s.at[pl.ds(k * STRIPE + t * ZROWS, ZROWS)])
            plsc.subcore_barrier()

            def sub(h, _):
                pltpu.sync_copy(
                    fg_ref.at[pl.ds(g * NP + sid * CH + h * SUB, SUB)],
                    fsub)

                def blk(j, _):
                    pltpu.sync_copy(fsub.at[pl.ds(j * 128, 128)],
                                    grid_s.at[rows2d.at[h * SUBB + j]],
                                    add=True)
                    return 0
                lax.fori_loop(0, SUBB, blk, 0)
                return 0
            lax.fori_loop(0, NSUB, sub, 0)
            plsc.subcore_barrier()

            for k in range(16):
                @pl.when(sid == k)
                def _():
                    pltpu.sync_copy(
                        grid_s.at[pl.ds(k * STRIPE, STRIPE)],
                        out_ref.at[pl.ds(g * ROWS_TOTAL + base + k * STRIPE,
                                         STRIPE)])
            return 0
        lax.fori_loop(0, NG, do_pass, 0)

    return scatter_kernel(fg, idx2d, zeros_src)


def _phase_c_body(i_ref, o_ref):
    v = i_ref[...]
    v = v.reshape(NG, 8 * Z, CG).transpose(0, 2, 1).reshape(COUT, 8 * Z)
    o_ref[...] = v.reshape(1, COUT, 8, Z)


def _phase_c(grid_g):
    return pl.pallas_call(
        _phase_c_body,
        grid=(B, X // 8),
        in_specs=[pl.BlockSpec((NG, 8 * Z * CG),
                               lambda b, i: (0, b * (X // 8) + i))],
        out_specs=pl.BlockSpec((1, COUT, 8, Z), lambda b, i: (b, 0, i, 0)),
        out_shape=jax.ShapeDtypeStruct((B, COUT, X, Z), jnp.float32),
    )(grid_g)


def kernel(coords, feats, kernel, stride):
    div = jnp.array([stride, stride, stride, 1], jnp.int32)
    coords_q = coords // div[None, :]
    coords_p = jnp.pad(coords_q, ((0, NP - N), (0, 0)))
    feats_p = jnp.pad(feats, ((0, NP - N), (0, 0)))
    fg, idx = _phase_a(coords_p, feats_p, kernel)
    fg3 = fg.reshape(NG, NP, CG).transpose(1, 0, 2).reshape(NP, COUT)
    grid = jnp.zeros((ROWS_TOTAL, COUT), jnp.float32).at[idx.reshape(NP)].add(
        fg3)
    grid_g = grid.reshape(ROWS_TOTAL, NG, CG).transpose(1, 0, 2)
    return _phase_c(grid_g.reshape(NG, ROWS_TOTAL * CG))
